# Initial kernel scaffold; baseline (speedup 1.0000x reference)
import jax
import jax.numpy as jnp
from jax.experimental import pallas as pl


def kernel(history_context_features, emb_weight):
    raise NotImplementedError("write your pallas kernel here")


# speedup vs baseline: 4.1460x; 4.1460x over previous
import functools

import jax
import jax.numpy as jnp
from jax import lax
from jax.experimental import pallas as pl
from jax.experimental.pallas import tpu as pltpu
from jax.experimental.pallas import tpu_sc as plsc

SITU_DIM = 100000
S = 4
D = 128
B = 4096
L = 50
P = B * L

NC, NS = 2, 16
NW = NC * NS
PPW = P // NW
C = 128
NCHUNK = PPW // C
NBUF = 5


def _sc_body(table_hbm, idxr_hbm, o0, o1, o2, o3, idx_v, buf, *sems):
    gsem = sems[:NBUF]
    ssem = sems[NBUF:]
    outs = (o0, o1, o2, o3)
    wid = lax.axis_index("s") * NC + lax.axis_index("c")

    pltpu.sync_copy(idxr_hbm.at[wid], idx_v)

    def g_start(i, j, b):
        pltpu.make_async_copy(
            table_hbm.at[idx_v.at[i * NCHUNK + j]], buf.at[b], gsem[b]
        ).start()

    def g_wait(b):
        pltpu.make_async_copy(
            table_hbm.at[idx_v.at[0]], buf.at[b], gsem[b]).wait()

    def s_start(i, j, b):
        pltpu.make_async_copy(
            buf.at[b], outs[i].at[pl.ds(wid * PPW + j * C, C)], ssem[b]
        ).start()

    def s_wait(i, b):
        pltpu.make_async_copy(
            buf.at[b], outs[i].at[pl.ds(0, C)], ssem[b]).wait()

    for i in range(S):
        for j in range(3):
            g_start(i, j, j)
        for j in range(2):
            g_wait(j)
            s_start(i, j, j)
            g_start(i, j + 3, j + 3)

        @pl.loop(2, 47, step=NBUF)
        def _(g):
            for k in range(NBUF):
                b = (2 + k) % NBUF
                j = g + k
                g_wait(b)
                s_start(i, j, b)
                s_wait(i, (b + 3) % NBUF)
                g_start(i, j + 3, (b + 3) % NBUF)

        for j in range(47, 50):
            b = j % NBUF
            g_wait(b)
            s_start(i, j, b)
            s_wait(i, (j - 2) % NBUF)
        s_wait(i, 48 % NBUF)
        s_wait(i, 49 % NBUF)


@jax.jit
def _run(table, idxr):
    out_sds = tuple(
        jax.ShapeDtypeStruct((P, D), jnp.float32) for _ in range(S))
    mesh = plsc.VectorSubcoreMesh(core_axis_name="c", subcore_axis_name="s")
    f = pl.kernel(
        _sc_body,
        out_type=out_sds,
        mesh=mesh,
        scratch_types=[
            pltpu.VMEM((S * NCHUNK, C), jnp.int32),
            pltpu.VMEM((NBUF, C, D), jnp.float32),
        ] + [pltpu.SemaphoreType.DMA] * (2 * NBUF),
    )
    return f(table, idxr)


def kernel(history_context_features, emb_weight):
    idx_t = history_context_features.reshape(P, S).T
    idxr = (idx_t.reshape(S, NW, NCHUNK, C)
            .transpose(1, 0, 2, 3)
            .reshape(NW, S * NCHUNK, C))
    outs = _run(emb_weight, idxr)
    return tuple(o.reshape(B, L, D) for o in outs)


# speedup vs baseline: 6.8322x; 1.6479x over previous
import jax
import jax.numpy as jnp
from jax import lax
from jax.experimental import pallas as pl
from jax.experimental.pallas import tpu as pltpu
from jax.experimental.pallas import tpu_sc as plsc

SITU_DIM = 100000
S = 4
D = 128
B = 4096
L = 50
LP = 56

NC, NS = 2, 16
NW = NC * NS
BPW = B // NW
NBC = 2
NCHUNK = BPW // NBC
NBUF = 4


def _sc_body(table_hbm, idxr_hbm, o0, o1, o2, o3, idx_v, buf, *sems):
    gsem = sems[:NBUF]
    ssem = sems[NBUF:]
    outs = (o0, o1, o2, o3)
    wid = lax.axis_index("s") * NC + lax.axis_index("c")

    pltpu.sync_copy(idxr_hbm.at[wid], idx_v)

    def g_start(i, j, b):
        for t in range(NBC):
            pltpu.make_async_copy(
                table_hbm.at[idx_v.at[i, j * NBC + t, pl.ds(0, L)]],
                buf.at[b, t], gsem[b]).start()

    def g_wait(b):
        for t in range(NBC):
            pltpu.make_async_copy(
                table_hbm.at[idx_v.at[0, 0, pl.ds(0, L)]],
                buf.at[b, t], gsem[b]).wait()

    def s_start(i, j, b):
        pltpu.make_async_copy(
            buf.at[b], outs[i].at[pl.ds(wid * BPW + j * NBC, NBC)], ssem[b]
        ).start()

    def s_wait(i, b):
        pltpu.make_async_copy(
            buf.at[b], outs[i].at[pl.ds(0, NBC)], ssem[b]).wait()

    for i in range(S):
        g_start(i, 0, 0)
        g_start(i, 1, 1)
        for j in range(2):
            g_wait(j)
            s_start(i, j, j)
            g_start(i, j + 2, j + 2)

        @pl.loop(2, 62, step=NBUF)
        def _(g):
            for k in range(NBUF):
                b = (2 + k) % NBUF
                j = g + k
                g_wait(b)
                s_start(i, j, b)
                s_wait(i, (b + 2) % NBUF)
                g_start(i, j + 2, (b + 2) % NBUF)

        for j in range(62, 64):
            b = j % NBUF
            g_wait(b)
            s_start(i, j, b)
            s_wait(i, (j - 2) % NBUF)
        s_wait(i, 62 % NBUF)
        s_wait(i, 63 % NBUF)


@jax.jit
def _run(table, idxr):
    out_sds = tuple(
        jax.ShapeDtypeStruct((B, L, D), jnp.float32) for _ in range(S))
    mesh = plsc.VectorSubcoreMesh(core_axis_name="c", subcore_axis_name="s")
    f = pl.kernel(
        _sc_body,
        out_type=out_sds,
        mesh=mesh,
        scratch_types=[
            pltpu.VMEM((S, BPW, LP), jnp.int32),
            pltpu.VMEM((NBUF, NBC, L, D), jnp.float32),
        ] + [pltpu.SemaphoreType.DMA] * (2 * NBUF),
    )
    return f(table, idxr)


def kernel(history_context_features, emb_weight):
    idx_t = jnp.transpose(history_context_features, (2, 0, 1))
    idx_p = jnp.pad(idx_t, ((0, 0), (0, 0), (0, LP - L)))
    idxr = jnp.transpose(idx_p.reshape(S, NW, BPW, LP), (1, 0, 2, 3))
    return _run(emb_weight, idxr)
